# baseline (speedup 1.0000x reference)
import jax
import jax.numpy as jnp
from jax import lax
from jax.experimental import pallas as pl
from jax.experimental.pallas import tpu as pltpu
from jax.experimental.pallas import tpu_sc as plsc

_DIM = 64
_CHUNK = 80
_NBUF = 2
_SCALE = 8.0
_L = 16
_NW = 32


def _sc_embed(tok_flat, table):
    n = tok_flat.shape[0]
    per_w = n // _NW
    nchunk = per_w // _CHUNK
    mesh = plsc.VectorSubcoreMesh(core_axis_name="c", subcore_axis_name="s")

    @pl.kernel(
        out_type=jax.ShapeDtypeStruct((n // 20, 20, _DIM), jnp.float32),
        mesh=mesh,
        compiler_params=pltpu.CompilerParams(use_tc_tiling_on_sc=False),
        scratch_types=[
            pltpu.VMEM((per_w,), jnp.int32),
            pltpu.VMEM((_NBUF, _CHUNK, _DIM), jnp.float32),
            pltpu.VMEM((_NBUF, _CHUNK // 20, 20, _DIM), jnp.float32),
            pltpu.SemaphoreType.DMA,
            pltpu.SemaphoreType.DMA,
            pltpu.SemaphoreType.DMA,
        ],
    )
    def k(tab_hbm, tok_hbm, out3d_hbm, idx_v, gbuf, obuf, sem_i, sem_g, sem_o):
        wid = lax.axis_index("s") * 2 + lax.axis_index("c")
        base = wid * per_w
        pltpu.async_copy(tok_hbm.at[pl.ds(base, per_w)], idx_v, sem_i).wait()

        def gather(kk, b):
            return pltpu.make_async_copy(
                tab_hbm.at[idx_v.at[pl.ds(kk * _CHUNK, _CHUNK)]],
                gbuf.at[b],
                sem_g,
            )

        def put(kk, b):
            return pltpu.make_async_copy(
                obuf.at[b],
                out3d_hbm.at[pl.ds((base + kk * _CHUNK) // 20, _CHUNK // 20)],
                sem_o,
            )

        for b in range(_NBUF):
            gather(b, b).start()

        @pl.loop(0, nchunk, step=_NBUF)
        def _(k0):
            for b in range(_NBUF):
                kk = k0 + b
                gather(kk, b).wait()

                @pl.when(kk >= _NBUF)
                def _():
                    put(kk - _NBUF, b).wait()

                @pl.loop(0, _CHUNK // 20)
                def _(q):
                    @pl.loop(0, 20)
                    def _(rr):
                        for c in range(0, _DIM, _L):
                            obuf.at[b, q, rr, pl.ds(c, _L)][...] = (
                                gbuf.at[b, q * 20 + rr, pl.ds(c, _L)][...]
                                * _SCALE
                            )

                put(kk, b).start()

                @pl.when(kk + _NBUF < nchunk)
                def _():
                    gather(kk + _NBUF, b).start()

        for b in range(_NBUF):
            put(nchunk - _NBUF + b, b).wait()

    return k(table, tok_flat)


def kernel(tokens, table):
    b, s = tokens.shape
    tok_flat = tokens.astype(jnp.int32).reshape(b * s)
    return _sc_embed(tok_flat, table)


# speedup vs baseline: 1.0230x; 1.0230x over previous
import jax
import jax.numpy as jnp
from jax import lax
from jax.experimental import pallas as pl
from jax.experimental.pallas import tpu as pltpu
from jax.experimental.pallas import tpu_sc as plsc

_DIM = 64
_CHUNK = 128
_NBUF = 2
_SCALE = 8.0
_L = 16
_NW = 32


def _sc_embed(tok_flat, table):
    n = tok_flat.shape[0]
    per_w = n // _NW
    nchunk = per_w // _CHUNK
    prow = _CHUNK // 2
    mesh = plsc.VectorSubcoreMesh(core_axis_name="c", subcore_axis_name="s")

    @pl.kernel(
        out_type=jax.ShapeDtypeStruct((n // 2, 2 * _DIM), jnp.float32),
        mesh=mesh,
        compiler_params=pltpu.CompilerParams(use_tc_tiling_on_sc=False),
        scratch_types=[
            pltpu.VMEM((per_w,), jnp.int32),
            pltpu.VMEM((_NBUF, _CHUNK, _DIM), jnp.float32),
            pltpu.VMEM((_NBUF, prow, 2 * _DIM), jnp.float32),
            pltpu.SemaphoreType.DMA,
            pltpu.SemaphoreType.DMA,
            pltpu.SemaphoreType.DMA,
        ],
    )
    def k(tab_hbm, tok_hbm, out_hbm, idx_v, gbuf, obuf, sem_i, sem_g, sem_o):
        wid = lax.axis_index("s") * 2 + lax.axis_index("c")
        base = wid * per_w
        pltpu.async_copy(tok_hbm.at[pl.ds(base, per_w)], idx_v, sem_i).wait()

        def gather(kk, b):
            return pltpu.make_async_copy(
                tab_hbm.at[idx_v.at[pl.ds(kk * _CHUNK, _CHUNK)]],
                gbuf.at[b],
                sem_g,
            )

        def put(kk, b):
            return pltpu.make_async_copy(
                obuf.at[b],
                out_hbm.at[pl.ds((base + kk * _CHUNK) // 2, prow)],
                sem_o,
            )

        for b in range(_NBUF):
            gather(b, b).start()

        @pl.loop(0, nchunk, step=_NBUF)
        def _(k0):
            for b in range(_NBUF):
                kk = k0 + b
                gather(kk, b).wait()

                @pl.when(kk >= _NBUF)
                def _():
                    put(kk - _NBUF, b).wait()

                @pl.loop(0, prow)
                def _(r):
                    for p in range(2):
                        for c in range(0, _DIM, _L):
                            obuf.at[b, r, pl.ds(p * _DIM + c, _L)][...] = (
                                gbuf.at[b, 2 * r + p, pl.ds(c, _L)][...]
                                * _SCALE
                            )

                put(kk, b).start()

                @pl.when(kk + _NBUF < nchunk)
                def _():
                    gather(kk + _NBUF, b).start()

        for b in range(_NBUF):
            put(nchunk - _NBUF + b, b).wait()

    return k(table, tok_flat)


def kernel(tokens, table):
    b, s = tokens.shape
    tok_flat = tokens.astype(jnp.int32).reshape(b * s)
    out = _sc_embed(tok_flat, table)
    return out.reshape(b, s, _DIM)
